# Initial kernel scaffold; baseline (speedup 1.0000x reference)
import jax
import jax.numpy as jnp
from jax.experimental import pallas as pl


def kernel(layer_depth, layer_type, ab_type, depth_table, depth_ln_w, depth_ln_b, type_table, type_ln_w, type_ln_b, ab_table, ab_ln_w, ab_ln_b, W, b):
    raise NotImplementedError("write your pallas kernel here")


# speedup vs baseline: 5.1834x; 5.1834x over previous
import functools

import jax
import jax.numpy as jnp
from jax import lax
from jax.experimental import pallas as pl
from jax.experimental.pallas import tpu as pltpu
from jax.experimental.pallas import tpu_sc as plsc

_EPS = 1e-5

_B = 16384
_D = 32
_P = 128
_NCOMBO = 128

_NC = 2
_NS = 16
_L = 16
_NW = _NC * _NS
_ROWS_W = _B // _NW
_CHROWS = 128
_CH = _ROWS_W // _CHROWS


def _table_body(dt, dw, db, tt, tw, tb, at_, aw, ab_, w_ref, b_ref, out):
    def ln(x, wv, bv):
        mu = jnp.mean(x, axis=-1, keepdims=True)
        var = jnp.mean((x - mu) ** 2, axis=-1, keepdims=True)
        return (x - mu) * lax.rsqrt(var + _EPS) * wv + bv

    dn = ln(dt[...], dw[...], db[...])
    tn = ln(tt[...], tw[...], tb[...])
    an = ln(at_[...], aw[...], ab_[...])

    w = w_ref[...]
    dims = (((1,), (1,)), ((), ()))
    pd = lax.dot_general(dn, w[:, :_D], dims, preferred_element_type=jnp.float32)
    pt = lax.dot_general(tn, w[:, _D:2 * _D], dims, preferred_element_type=jnp.float32)
    pa = lax.dot_general(an, w[:, 2 * _D:], dims, preferred_element_type=jnp.float32)

    ic = lax.broadcasted_iota(jnp.int32, (_NCOMBO, _D), 0)
    iv = lax.broadcasted_iota(jnp.int32, (_NCOMBO, _D), 1)
    e_d = (ic // 4 == iv).astype(jnp.float32)
    hd = jnp.dot(e_d, pd, preferred_element_type=jnp.float32)

    cid = lax.broadcasted_iota(jnp.int32, (_NCOMBO, _P), 0)
    tbit = ((cid >> 1) & 1).astype(jnp.float32)
    abit = (cid & 1).astype(jnp.float32)
    h = (hd
         + pt[0:1, :] + tbit * (pt[1:2, :] - pt[0:1, :])
         + pa[0:1, :] + abit * (pa[1:2, :] - pa[0:1, :])
         + b_ref[...])
    out[...] = h * (1.0 / (1.0 + jnp.exp(-h)))


_table_call = pl.pallas_call(
    _table_body,
    out_shape=jax.ShapeDtypeStruct((_NCOMBO, _P), jnp.float32),
)


def _gather_body(d_hbm, t_hbm, a_hbm, table_hbm, out_hbm,
                 d_v, t_v, a_v, idx_v, rows_v, gsem):
    wid = lax.axis_index("s") * _NC + lax.axis_index("c")
    base = wid * _CH
    pltpu.sync_copy(d_hbm.at[pl.ds(base, _CH)], d_v)
    pltpu.sync_copy(t_hbm.at[pl.ds(base, _CH)], t_v)
    pltpu.sync_copy(a_hbm.at[pl.ds(base, _CH)], a_v)
    for j in range(_CH):
        for i in range(_CHROWS // _L):
            sl = pl.ds(i * _L, _L)
            idx_v[j, sl] = d_v[j, sl] * 4 + t_v[j, sl] * 2 + a_v[j, sl]
    copies = [
        pltpu.async_copy(table_hbm.at[idx_v.at[j]],
                         rows_v.at[pl.ds(j * _CHROWS, _CHROWS)], gsem)
        for j in range(_CH)
    ]
    for c in copies:
        c.wait()
    pltpu.sync_copy(rows_v, out_hbm.at[pl.ds(wid * _ROWS_W, _ROWS_W)])


@functools.cache
def _make_gather_call():
    mesh = plsc.VectorSubcoreMesh(core_axis_name="c", subcore_axis_name="s",
                                  num_cores=_NC, num_subcores=_NS)
    return pl.kernel(
        _gather_body,
        mesh=mesh,
        out_type=jax.ShapeDtypeStruct((_B, _P), jnp.float32),
        scratch_types=[
            pltpu.VMEM((_CH, _CHROWS), jnp.int32),
            pltpu.VMEM((_CH, _CHROWS), jnp.int32),
            pltpu.VMEM((_CH, _CHROWS), jnp.int32),
            pltpu.VMEM((_CH, _CHROWS), jnp.int32),
            pltpu.VMEM((_ROWS_W, _P), jnp.float32),
            pltpu.SemaphoreType.DMA,
        ],
    )


def kernel(layer_depth, layer_type, ab_type, depth_table, depth_ln_w, depth_ln_b,
           type_table, type_ln_w, type_ln_b, ab_table, ab_ln_w, ab_ln_b, W, b):
    table = _table_call(
        depth_table, depth_ln_w.reshape(1, _D), depth_ln_b.reshape(1, _D),
        type_table, type_ln_w.reshape(1, _D), type_ln_b.reshape(1, _D),
        ab_table, ab_ln_w.reshape(1, _D), ab_ln_b.reshape(1, _D),
        W, b.reshape(1, _P),
    )
    d2 = layer_depth.astype(jnp.int32).reshape(_NW * _CH, _CHROWS)
    t2 = layer_type.astype(jnp.int32).reshape(_NW * _CH, _CHROWS)
    a2 = ab_type.astype(jnp.int32).reshape(_NW * _CH, _CHROWS)
    return _make_gather_call()(d2, t2, a2, table)


# speedup vs baseline: 5.4403x; 1.0496x over previous
import functools

import jax
import jax.numpy as jnp
from jax import lax
from jax.experimental import pallas as pl
from jax.experimental.pallas import tpu as pltpu
from jax.experimental.pallas import tpu_sc as plsc

_EPS = 1e-5

_B = 16384
_D = 32
_P = 128
_NCOMBO = 128

_NC = 2
_NS = 16
_L = 16
_NW = _NC * _NS
_ROWS_W = _B // _NW
_CHROWS = 128
_CH = _ROWS_W // _CHROWS


def _table_body(dt, dw, db, tt, tw, tb, at_, aw, ab_, w_ref, b_ref, out):
    def ln(x, wv, bv):
        mu = jnp.mean(x, axis=-1, keepdims=True)
        var = jnp.mean((x - mu) ** 2, axis=-1, keepdims=True)
        return (x - mu) * lax.rsqrt(var + _EPS) * wv + bv

    dn = ln(dt[...], dw[...], db[...])
    tn = ln(tt[...], tw[...], tb[...])
    an = ln(at_[...], aw[...], ab_[...])

    w = w_ref[...]
    dims = (((1,), (1,)), ((), ()))
    pd = lax.dot_general(dn, w[:, :_D], dims, preferred_element_type=jnp.float32)
    pt = lax.dot_general(tn, w[:, _D:2 * _D], dims, preferred_element_type=jnp.float32)
    pa = lax.dot_general(an, w[:, 2 * _D:], dims, preferred_element_type=jnp.float32)

    ic = lax.broadcasted_iota(jnp.int32, (_NCOMBO, _D), 0)
    iv = lax.broadcasted_iota(jnp.int32, (_NCOMBO, _D), 1)
    e_d = (ic // 4 == iv).astype(jnp.float32)
    hd = jnp.dot(e_d, pd, preferred_element_type=jnp.float32)

    cid = lax.broadcasted_iota(jnp.int32, (_NCOMBO, _P), 0)
    tbit = ((cid >> 1) & 1).astype(jnp.float32)
    abit = (cid & 1).astype(jnp.float32)
    h = (hd
         + pt[0:1, :] + tbit * (pt[1:2, :] - pt[0:1, :])
         + pa[0:1, :] + abit * (pa[1:2, :] - pa[0:1, :])
         + b_ref[...])
    out[...] = h * (1.0 / (1.0 + jnp.exp(-h)))


_table_call = pl.pallas_call(
    _table_body,
    out_shape=jax.ShapeDtypeStruct((_NCOMBO, _P), jnp.float32),
)


def _gather_body(d_hbm, t_hbm, a_hbm, table_hbm, out_hbm,
                 d_v, t_v, a_v, idx_v, rows_v, isem, gsems, ssem):
    wid = lax.axis_index("s") * _NC + lax.axis_index("c")
    base = wid * _ROWS_W
    ic = [
        pltpu.async_copy(d_hbm.at[pl.ds(base, _ROWS_W)], d_v, isem),
        pltpu.async_copy(t_hbm.at[pl.ds(base, _ROWS_W)], t_v, isem),
        pltpu.async_copy(a_hbm.at[pl.ds(base, _ROWS_W)], a_v, isem),
    ]
    for c in ic:
        c.wait()
    for i in range(_ROWS_W // _L):
        sl = pl.ds(i * _L, _L)
        idx_v[sl] = d_v[sl] * 4 + t_v[sl] * 2 + a_v[sl]
    gathers = [
        pltpu.async_copy(table_hbm.at[idx_v.at[pl.ds(j * _CHROWS, _CHROWS)]],
                         rows_v.at[pl.ds(j * _CHROWS, _CHROWS)], gsems.at[j])
        for j in range(_CH)
    ]
    stores = []
    for j in range(_CH):
        gathers[j].wait()
        stores.append(
            pltpu.async_copy(rows_v.at[pl.ds(j * _CHROWS, _CHROWS)],
                             out_hbm.at[pl.ds(base + j * _CHROWS, _CHROWS)],
                             ssem))
    for s in stores:
        s.wait()


@functools.cache
def _make_gather_call():
    mesh = plsc.VectorSubcoreMesh(core_axis_name="c", subcore_axis_name="s",
                                  num_cores=_NC, num_subcores=_NS)
    return pl.kernel(
        _gather_body,
        mesh=mesh,
        out_type=jax.ShapeDtypeStruct((_B, _P), jnp.float32),
        scratch_types=[
            pltpu.VMEM((_ROWS_W,), jnp.int32),
            pltpu.VMEM((_ROWS_W,), jnp.int32),
            pltpu.VMEM((_ROWS_W,), jnp.int32),
            pltpu.VMEM((_ROWS_W,), jnp.int32),
            pltpu.VMEM((_ROWS_W, _P), jnp.float32),
            pltpu.SemaphoreType.DMA,
            pltpu.SemaphoreType.DMA((_CH,)),
            pltpu.SemaphoreType.DMA,
        ],
    )


def kernel(layer_depth, layer_type, ab_type, depth_table, depth_ln_w, depth_ln_b,
           type_table, type_ln_w, type_ln_b, ab_table, ab_ln_w, ab_ln_b, W, b):
    table = _table_call(
        depth_table, depth_ln_w.reshape(1, _D), depth_ln_b.reshape(1, _D),
        type_table, type_ln_w.reshape(1, _D), type_ln_b.reshape(1, _D),
        ab_table, ab_ln_w.reshape(1, _D), ab_ln_b.reshape(1, _D),
        W, b.reshape(1, _P),
    )
    return _make_gather_call()(layer_depth.astype(jnp.int32),
                               layer_type.astype(jnp.int32),
                               ab_type.astype(jnp.int32), table)


# speedup vs baseline: 7.2172x; 1.3266x over previous
import functools

import jax
import jax.numpy as jnp
from jax import lax
from jax.experimental import pallas as pl
from jax.experimental.pallas import tpu as pltpu
from jax.experimental.pallas import tpu_sc as plsc

_EPS = 1e-5

_B = 16384
_D = 32
_P = 128
_NCOMBO = 128

_NC = 2
_NS = 16
_L = 16
_NW = _NC * _NS
_ROWS_W = _B // _NW
_CHROWS = 128
_CH = _ROWS_W // _CHROWS


def _table_body(dt, dw, db, tt, tw, tb, at_, aw, ab_, w_ref, b_ref, out):
    def ln(x, wv, bv):
        mu = jnp.mean(x, axis=-1, keepdims=True)
        var = jnp.mean((x - mu) ** 2, axis=-1, keepdims=True)
        return (x - mu) * lax.rsqrt(var + _EPS) * wv + bv

    dn = ln(dt[...], dw[...], db[...])
    tn = ln(tt[...], tw[...], tb[...])
    an = ln(at_[...], aw[...], ab_[...])

    w = w_ref[...]
    dims = (((1,), (1,)), ((), ()))
    pd = lax.dot_general(dn, w[:, :_D], dims, preferred_element_type=jnp.float32)
    pt = lax.dot_general(tn, w[:, _D:2 * _D], dims, preferred_element_type=jnp.float32)
    pa = lax.dot_general(an, w[:, 2 * _D:], dims, preferred_element_type=jnp.float32)

    ic = lax.broadcasted_iota(jnp.int32, (_NCOMBO, _D), 0)
    iv = lax.broadcasted_iota(jnp.int32, (_NCOMBO, _D), 1)
    e_d = (ic // 4 == iv).astype(jnp.float32)
    hd = jnp.dot(e_d, pd, preferred_element_type=jnp.float32)

    cid = lax.broadcasted_iota(jnp.int32, (_NCOMBO, _P), 0)
    tbit = ((cid >> 1) & 1).astype(jnp.float32)
    abit = (cid & 1).astype(jnp.float32)
    h = (hd
         + pt[0:1, :] + tbit * (pt[1:2, :] - pt[0:1, :])
         + pa[0:1, :] + abit * (pa[1:2, :] - pa[0:1, :])
         + b_ref[...])
    out[...] = h * (1.0 / (1.0 + jnp.exp(-h)))


_table_call = pl.pallas_call(
    _table_body,
    out_shape=jax.ShapeDtypeStruct((_NCOMBO, _P), jnp.float32),
)


def _gather_body(d_hbm, t_hbm, a_hbm, table_hbm, out_hbm,
                 d_v, t_v, a_v, idx_v, rows_v, table_sh, isem, tsem, gsems, ssem):
    sid = lax.axis_index("s")
    wid = sid * _NC + lax.axis_index("c")
    base = wid * _ROWS_W
    ic = [
        pltpu.async_copy(d_hbm.at[pl.ds(base, _ROWS_W)], d_v, isem),
        pltpu.async_copy(t_hbm.at[pl.ds(base, _ROWS_W)], t_v, isem),
        pltpu.async_copy(a_hbm.at[pl.ds(base, _ROWS_W)], a_v, isem),
    ]
    @pl.when(sid == 0)
    def _stage_table():
        tv = rows_v.at[pl.ds(0, _NCOMBO)]
        pltpu.async_copy(table_hbm, tv, tsem).wait()
        pltpu.sync_copy(tv, table_sh)
    for c in ic:
        c.wait()
    for i in range(_ROWS_W // _L):
        sl = pl.ds(i * _L, _L)
        idx_v[sl] = d_v[sl] * 4 + t_v[sl] * 2 + a_v[sl]
    plsc.subcore_barrier()
    gathers = [
        pltpu.async_copy(table_sh.at[idx_v.at[pl.ds(j * _CHROWS, _CHROWS)]],
                         rows_v.at[pl.ds(j * _CHROWS, _CHROWS)], gsems.at[j])
        for j in range(_CH)
    ]
    stores = []
    for j in range(_CH):
        gathers[j].wait()
        stores.append(
            pltpu.async_copy(rows_v.at[pl.ds(j * _CHROWS, _CHROWS)],
                             out_hbm.at[pl.ds(base + j * _CHROWS, _CHROWS)],
                             ssem))
    for s in stores:
        s.wait()


@functools.cache
def _make_gather_call():
    mesh = plsc.VectorSubcoreMesh(core_axis_name="c", subcore_axis_name="s",
                                  num_cores=_NC, num_subcores=_NS)
    return pl.kernel(
        _gather_body,
        mesh=mesh,
        out_type=jax.ShapeDtypeStruct((_B, _P), jnp.float32),
        scratch_types=[
            pltpu.VMEM((_ROWS_W,), jnp.int32),
            pltpu.VMEM((_ROWS_W,), jnp.int32),
            pltpu.VMEM((_ROWS_W,), jnp.int32),
            pltpu.VMEM((_ROWS_W,), jnp.int32),
            pltpu.VMEM((_ROWS_W, _P), jnp.float32),
            pltpu.VMEM_SHARED((_NCOMBO, _P), jnp.float32),
            pltpu.SemaphoreType.DMA,
            pltpu.SemaphoreType.DMA,
            pltpu.SemaphoreType.DMA((_CH,)),
            pltpu.SemaphoreType.DMA,
        ],
    )


def kernel(layer_depth, layer_type, ab_type, depth_table, depth_ln_w, depth_ln_b,
           type_table, type_ln_w, type_ln_b, ab_table, ab_ln_w, ab_ln_b, W, b):
    table = _table_call(
        depth_table, depth_ln_w.reshape(1, _D), depth_ln_b.reshape(1, _D),
        type_table, type_ln_w.reshape(1, _D), type_ln_b.reshape(1, _D),
        ab_table, ab_ln_w.reshape(1, _D), ab_ln_b.reshape(1, _D),
        W, b.reshape(1, _P),
    )
    return _make_gather_call()(layer_depth.astype(jnp.int32),
                               layer_type.astype(jnp.int32),
                               ab_type.astype(jnp.int32), table)


# speedup vs baseline: 7.2191x; 1.0003x over previous
import functools

import jax
import jax.numpy as jnp
from jax import lax
from jax.experimental import pallas as pl
from jax.experimental.pallas import tpu as pltpu
from jax.experimental.pallas import tpu_sc as plsc

_EPS = 1e-5

_B = 16384
_D = 32
_P = 128
_NCOMBO = 128

_NC = 2
_NS = 16
_L = 16
_NW = _NC * _NS
_ROWS_W = _B // _NW
_CHROWS = 128
_CH = _ROWS_W // _CHROWS


def _table_body(d_ref, t_ref, a_ref, dt, dw, db, tt, tw, tb, at_, aw, ab_,
                w_ref, b_ref, out, combo_ref):
    combo_ref[...] = d_ref[...] * 4 + t_ref[...] * 2 + a_ref[...]
    def ln(x, wv, bv):
        mu = jnp.mean(x, axis=-1, keepdims=True)
        var = jnp.mean((x - mu) ** 2, axis=-1, keepdims=True)
        return (x - mu) * lax.rsqrt(var + _EPS) * wv + bv

    dn = ln(dt[...], dw[...], db[...])
    tn = ln(tt[...], tw[...], tb[...])
    an = ln(at_[...], aw[...], ab_[...])

    w = w_ref[...]
    dims = (((1,), (1,)), ((), ()))
    pd = lax.dot_general(dn, w[:, :_D], dims, preferred_element_type=jnp.float32)
    pt = lax.dot_general(tn, w[:, _D:2 * _D], dims, preferred_element_type=jnp.float32)
    pa = lax.dot_general(an, w[:, 2 * _D:], dims, preferred_element_type=jnp.float32)

    ic = lax.broadcasted_iota(jnp.int32, (_NCOMBO, _D), 0)
    iv = lax.broadcasted_iota(jnp.int32, (_NCOMBO, _D), 1)
    e_d = (ic // 4 == iv).astype(jnp.float32)
    hd = jnp.dot(e_d, pd, preferred_element_type=jnp.float32)

    cid = lax.broadcasted_iota(jnp.int32, (_NCOMBO, _P), 0)
    tbit = ((cid >> 1) & 1).astype(jnp.float32)
    abit = (cid & 1).astype(jnp.float32)
    h = (hd
         + pt[0:1, :] + tbit * (pt[1:2, :] - pt[0:1, :])
         + pa[0:1, :] + abit * (pa[1:2, :] - pa[0:1, :])
         + b_ref[...])
    out[...] = h * (1.0 / (1.0 + jnp.exp(-h)))


_table_call = pl.pallas_call(
    _table_body,
    out_shape=(
        jax.ShapeDtypeStruct((_NCOMBO, _P), jnp.float32),
        jax.ShapeDtypeStruct((_B // _CHROWS, _CHROWS), jnp.int32),
    ),
)


def _gather_body(combo_hbm, table_hbm, out_hbm,
                 idx_v, rows_v, table_sh, isem, tsem, gsems, ssem):
    sid = lax.axis_index("s")
    wid = sid * _NC + lax.axis_index("c")
    base = wid * _ROWS_W
    ic = pltpu.async_copy(combo_hbm.at[pl.ds(wid * _CH, _CH)], idx_v, isem)
    @pl.when(sid == 0)
    def _stage_table():
        tv = rows_v.at[pl.ds(0, _NCOMBO)]
        pltpu.async_copy(table_hbm, tv, tsem).wait()
        pltpu.sync_copy(tv, table_sh)
    ic.wait()
    plsc.subcore_barrier()
    gathers = [
        pltpu.async_copy(table_sh.at[idx_v.at[j]],
                         rows_v.at[pl.ds(j * _CHROWS, _CHROWS)], gsems.at[j])
        for j in range(_CH)
    ]
    stores = []
    for j in range(_CH):
        gathers[j].wait()
        stores.append(
            pltpu.async_copy(rows_v.at[pl.ds(j * _CHROWS, _CHROWS)],
                             out_hbm.at[pl.ds(base + j * _CHROWS, _CHROWS)],
                             ssem))
    for s in stores:
        s.wait()


@functools.cache
def _make_gather_call():
    mesh = plsc.VectorSubcoreMesh(core_axis_name="c", subcore_axis_name="s",
                                  num_cores=_NC, num_subcores=_NS)
    return pl.kernel(
        _gather_body,
        mesh=mesh,
        out_type=jax.ShapeDtypeStruct((_B, _P), jnp.float32),
        scratch_types=[
            pltpu.VMEM((_CH, _CHROWS), jnp.int32),
            pltpu.VMEM((_ROWS_W, _P), jnp.float32),
            pltpu.VMEM_SHARED((_NCOMBO, _P), jnp.float32),
            pltpu.SemaphoreType.DMA,
            pltpu.SemaphoreType.DMA,
            pltpu.SemaphoreType.DMA((_CH,)),
            pltpu.SemaphoreType.DMA,
        ],
    )


def kernel(layer_depth, layer_type, ab_type, depth_table, depth_ln_w, depth_ln_b,
           type_table, type_ln_w, type_ln_b, ab_table, ab_ln_w, ab_ln_b, W, b):
    nrow = _B // _CHROWS
    table, combo = _table_call(
        layer_depth.astype(jnp.int32).reshape(nrow, _CHROWS),
        layer_type.astype(jnp.int32).reshape(nrow, _CHROWS),
        ab_type.astype(jnp.int32).reshape(nrow, _CHROWS),
        depth_table, depth_ln_w.reshape(1, _D), depth_ln_b.reshape(1, _D),
        type_table, type_ln_w.reshape(1, _D), type_ln_b.reshape(1, _D),
        ab_table, ab_ln_w.reshape(1, _D), ab_ln_b.reshape(1, _D),
        W, b.reshape(1, _P),
    )
    return _make_gather_call()(combo, table)


# speedup vs baseline: 7.4995x; 1.0388x over previous
import functools

import jax
import jax.numpy as jnp
from jax import lax
from jax.experimental import pallas as pl
from jax.experimental.pallas import tpu as pltpu
from jax.experimental.pallas import tpu_sc as plsc

_EPS = 1e-5

_B = 16384
_D = 32
_P = 128
_NCOMBO = 128

_NC = 2
_NS = 16
_L = 16
_NW = _NC * _NS
_ROWS_W = _B // _NW
_CHROWS = 128
_CH = _ROWS_W // _CHROWS


def _table_body(d_ref, t_ref, a_ref, dt, dw, db, tt, tw, tb, at_, aw, ab_,
                w_ref, b_ref, out, combo_ref):
    combo_ref[...] = d_ref[...] * 4 + t_ref[...] * 2 + a_ref[...]
    def ln(x, wv, bv):
        mu = jnp.mean(x, axis=-1, keepdims=True)
        var = jnp.mean((x - mu) ** 2, axis=-1, keepdims=True)
        return (x - mu) * lax.rsqrt(var + _EPS) * wv + bv

    dn = ln(dt[...], dw[...], db[...])
    tn = ln(tt[...], tw[...], tb[...])
    an = ln(at_[...], aw[...], ab_[...])

    w = w_ref[...]
    dims = (((1,), (1,)), ((), ()))
    pd = lax.dot_general(dn, w[:, :_D], dims, preferred_element_type=jnp.float32)
    pt = lax.dot_general(tn, w[:, _D:2 * _D], dims, preferred_element_type=jnp.float32)
    pa = lax.dot_general(an, w[:, 2 * _D:], dims, preferred_element_type=jnp.float32)

    ic = lax.broadcasted_iota(jnp.int32, (_NCOMBO, _D), 0)
    iv = lax.broadcasted_iota(jnp.int32, (_NCOMBO, _D), 1)
    e_d = (ic // 4 == iv).astype(jnp.float32)
    hd = jnp.dot(e_d, pd, preferred_element_type=jnp.float32)

    cid = lax.broadcasted_iota(jnp.int32, (_NCOMBO, _P), 0)
    tbit = ((cid >> 1) & 1).astype(jnp.float32)
    abit = (cid & 1).astype(jnp.float32)
    h = (hd
         + pt[0:1, :] + tbit * (pt[1:2, :] - pt[0:1, :])
         + pa[0:1, :] + abit * (pa[1:2, :] - pa[0:1, :])
         + b_ref[...])
    out[...] = h * (1.0 / (1.0 + jnp.exp(-h)))


_table_call = pl.pallas_call(
    _table_body,
    out_shape=(
        jax.ShapeDtypeStruct((_NCOMBO, _P), jnp.float32),
        jax.ShapeDtypeStruct((_B // _CHROWS, _CHROWS), jnp.int32),
    ),
)


def _gather_body(combo_hbm, table_hbm, out_hbm,
                 idx_v, rows_v, table_sh, isem, tsem, gsems, ssem):
    sid = lax.axis_index("s")
    wid = sid * _NC + lax.axis_index("c")
    base = wid * _ROWS_W
    ic = pltpu.async_copy(combo_hbm.at[pl.ds(wid * _CH, _CH)], idx_v, isem)
    nstage = _NCOMBO // _NS
    tv = rows_v.at[pl.ds(0, nstage)]
    trows = pl.ds(sid * nstage, nstage)
    pltpu.async_copy(table_hbm.at[trows], tv, tsem).wait()
    pltpu.sync_copy(tv, table_sh.at[trows])
    ic.wait()
    plsc.subcore_barrier()
    gathers = [
        pltpu.async_copy(table_sh.at[idx_v.at[j]],
                         rows_v.at[pl.ds(j * _CHROWS, _CHROWS)], gsems.at[j])
        for j in range(_CH)
    ]
    stores = []
    for j in range(_CH):
        gathers[j].wait()
        stores.append(
            pltpu.async_copy(rows_v.at[pl.ds(j * _CHROWS, _CHROWS)],
                             out_hbm.at[pl.ds(base + j * _CHROWS, _CHROWS)],
                             ssem))
    for s in stores:
        s.wait()


@functools.cache
def _make_gather_call():
    mesh = plsc.VectorSubcoreMesh(core_axis_name="c", subcore_axis_name="s",
                                  num_cores=_NC, num_subcores=_NS)
    return pl.kernel(
        _gather_body,
        mesh=mesh,
        out_type=jax.ShapeDtypeStruct((_B, _P), jnp.float32),
        scratch_types=[
            pltpu.VMEM((_CH, _CHROWS), jnp.int32),
            pltpu.VMEM((_ROWS_W, _P), jnp.float32),
            pltpu.VMEM_SHARED((_NCOMBO, _P), jnp.float32),
            pltpu.SemaphoreType.DMA,
            pltpu.SemaphoreType.DMA,
            pltpu.SemaphoreType.DMA((_CH,)),
            pltpu.SemaphoreType.DMA,
        ],
    )


def kernel(layer_depth, layer_type, ab_type, depth_table, depth_ln_w, depth_ln_b,
           type_table, type_ln_w, type_ln_b, ab_table, ab_ln_w, ab_ln_b, W, b):
    nrow = _B // _CHROWS
    table, combo = _table_call(
        layer_depth.astype(jnp.int32).reshape(nrow, _CHROWS),
        layer_type.astype(jnp.int32).reshape(nrow, _CHROWS),
        ab_type.astype(jnp.int32).reshape(nrow, _CHROWS),
        depth_table, depth_ln_w.reshape(1, _D), depth_ln_b.reshape(1, _D),
        type_table, type_ln_w.reshape(1, _D), type_ln_b.reshape(1, _D),
        ab_table, ab_ln_w.reshape(1, _D), ab_ln_b.reshape(1, _D),
        W, b.reshape(1, _P),
    )
    return _make_gather_call()(combo, table)
